# baseline (speedup 1.0000x reference)
import functools

import jax
import jax.numpy as jnp
from jax.experimental import pallas as pl
from jax.experimental.pallas import tpu as pltpu


def _gat_body(x_ref, wfct_ref, acomb_ref, wproj_ref, bcol_ref, out_ref,
              *, H, D):
    f32 = jnp.float32
    bf16 = jnp.bfloat16
    xb = x_ref[0]
    wfct = wfct_ref[...]
    acomb = acomb_ref[...]
    wproj = wproj_ref[...]

    feat = jax.lax.dot_general(xb, wfct, (((0,), (0,)), ((), ())),
                               preferred_element_type=f32)
    featb = feat.astype(bf16)
    lr = jax.lax.dot_general(featb, acomb, (((1,), (0,)), ((), ())),
                             preferred_element_type=f32)
    lrT = jax.lax.dot_general(acomb, featb, (((0,), (1,)), ((), ())),
                              preferred_element_type=f32)
    lr_bf = lr.astype(bf16)
    lrT_bf = lrT.astype(bf16)
    el_maxs = jnp.max(lr, axis=0, keepdims=True)

    acc = None
    for h in range(H):
        el_col = lr_bf[:, h:h + 1]
        er_row = lrT_bf[H + h:H + h + 1, :]
        a = el_col + er_row
        e = jnp.maximum(a, 0.2 * a)
        t = el_maxs[0, h] + lrT[H + h:H + h + 1, :]
        m = jnp.maximum(t, 0.2 * t).astype(bf16)
        p = jnp.exp(e - m)
        s = jnp.sum(p, axis=0, keepdims=True)
        r = (1.0 / s.astype(f32)).astype(bf16)
        alpha = p * r
        fh = featb[:, h * D:(h + 1) * D]
        rst_h = jax.lax.dot_general(alpha, fh, (((0,), (0,)), ((), ())),
                                    preferred_element_type=f32)
        out_h = jax.lax.dot_general(wproj[:, h * D:(h + 1) * D],
                                    rst_h.astype(bf16),
                                    (((1,), (1,)), ((), ())),
                                    preferred_element_type=f32)
        acc = out_h if acc is None else acc + out_h

    out_ref[0] = acc + bcol_ref[...]


def kernel(x, W_fc, attn_l, attn_r, bias_gat, W_proj, b_proj):
    B, W, F = x.shape
    H, D = attn_l.shape
    HD = H * D

    f32 = jnp.float32
    bf16 = jnp.bfloat16
    Wfc_T = W_fc.astype(f32).T.astype(bf16)
    eye = jnp.eye(H, dtype=f32)
    Al = (attn_l.astype(f32)[:, :, None] * eye[:, None, :]).reshape(HD, H)
    Ar = (attn_r.astype(f32)[:, :, None] * eye[:, None, :]).reshape(HD, H)
    A_comb = jnp.concatenate([Al, Ar], axis=1).astype(bf16)
    bcol = (W_proj.astype(f32) @ bias_gat.astype(f32)
            + b_proj.astype(f32)).reshape(W, 1)

    body = functools.partial(_gat_body, H=H, D=D)
    out = pl.pallas_call(
        body,
        grid=(B,),
        in_specs=[
            pl.BlockSpec((1, W, F), lambda b: (b, 0, 0)),
            pl.BlockSpec((W, HD), lambda b: (0, 0)),
            pl.BlockSpec((HD, 2 * H), lambda b: (0, 0)),
            pl.BlockSpec((W, HD), lambda b: (0, 0)),
            pl.BlockSpec((W, 1), lambda b: (0, 0)),
        ],
        out_specs=pl.BlockSpec((1, W, F), lambda b: (b, 0, 0)),
        out_shape=jax.ShapeDtypeStruct((B, W, F), f32),
        compiler_params=pltpu.CompilerParams(
            dimension_semantics=("parallel",)),
    )(x.astype(bf16), Wfc_T, A_comb, W_proj.astype(bf16), bcol)
    return out


# speedup vs baseline: 1.4380x; 1.4380x over previous
import functools

import jax
import jax.numpy as jnp
from jax.experimental import pallas as pl
from jax.experimental.pallas import tpu as pltpu


def _gat_body(node_ref, wfct_ref, acomb_ref, wproj_ref, bcol_ref, out_ref,
              *, H, D, NB):
    f32 = jnp.float32
    bf16 = jnp.bfloat16
    wfct = wfct_ref[...]
    acomb = acomb_ref[...]
    wproj = wproj_ref[...]
    bcol = bcol_ref[...]

    for j in range(NB):
        nb = node_ref[j]
        feat = jax.lax.dot_general(nb, wfct, (((1,), (0,)), ((), ())),
                                   preferred_element_type=f32)
        featb = feat.astype(bf16)
        lr = jax.lax.dot_general(featb, acomb, (((1,), (0,)), ((), ())),
                                 preferred_element_type=f32)
        lrT = jax.lax.dot_general(acomb, featb, (((0,), (1,)), ((), ())),
                                  preferred_element_type=f32)
        lr_bf = lr.astype(bf16)
        lrT_bf = lrT.astype(bf16)
        el_maxs = jnp.max(lr, axis=0, keepdims=True)

        acc = None
        for h in range(H):
            el_col = lr_bf[:, h:h + 1]
            er_row = lrT_bf[H + h:H + h + 1, :]
            a = el_col + er_row
            e = jnp.maximum(a, 0.2 * a)
            t = el_maxs[0, h] + lrT[H + h:H + h + 1, :]
            m = jnp.maximum(t, 0.2 * t).astype(bf16)
            p = jnp.exp(e - m)
            s = jnp.sum(p, axis=0, keepdims=True)
            r = (1.0 / s.astype(f32)).astype(bf16)
            alpha = p * r
            fh = featb[:, h * D:(h + 1) * D]
            rst_h = jax.lax.dot_general(alpha, fh, (((0,), (0,)), ((), ())),
                                        preferred_element_type=f32)
            out_h = jax.lax.dot_general(wproj[:, h * D:(h + 1) * D],
                                        rst_h.astype(bf16),
                                        (((1,), (1,)), ((), ())),
                                        preferred_element_type=f32)
            acc = out_h if acc is None else acc + out_h

        out_ref[j] = acc + bcol


def kernel(x, W_fc, attn_l, attn_r, bias_gat, W_proj, b_proj):
    B, W, F = x.shape
    H, D = attn_l.shape
    HD = H * D

    f32 = jnp.float32
    bf16 = jnp.bfloat16
    Wfc_T = W_fc.astype(f32).T.astype(bf16)
    eye = jnp.eye(H, dtype=f32)
    Al = (attn_l.astype(f32)[:, :, None] * eye[:, None, :]).reshape(HD, H)
    Ar = (attn_r.astype(f32)[:, :, None] * eye[:, None, :]).reshape(HD, H)
    A_comb = jnp.concatenate([Al, Ar], axis=1).astype(bf16)
    bcol = (W_proj.astype(f32) @ bias_gat.astype(f32)
            + b_proj.astype(f32)).reshape(W, 1)

    NB = 8
    node = jnp.transpose(x.astype(bf16), (0, 2, 1))
    body = functools.partial(_gat_body, H=H, D=D, NB=NB)
    out = pl.pallas_call(
        body,
        grid=(B // NB,),
        in_specs=[
            pl.BlockSpec((NB, F, W), lambda b: (b, 0, 0)),
            pl.BlockSpec((W, HD), lambda b: (0, 0)),
            pl.BlockSpec((HD, 2 * H), lambda b: (0, 0)),
            pl.BlockSpec((W, HD), lambda b: (0, 0)),
            pl.BlockSpec((W, 1), lambda b: (0, 0)),
        ],
        out_specs=pl.BlockSpec((NB, W, F), lambda b: (b, 0, 0)),
        out_shape=jax.ShapeDtypeStruct((B, W, F), f32),
        compiler_params=pltpu.CompilerParams(
            dimension_semantics=("parallel",)),
    )(node, Wfc_T, A_comb, W_proj.astype(bf16), bcol)
    return out


# speedup vs baseline: 1.9862x; 1.3812x over previous
import functools

import jax
import jax.numpy as jnp
from jax.experimental import pallas as pl
from jax.experimental.pallas import tpu as pltpu


def _gat_body(node_ref, wfcte_ref, wproj_ref, bcol_ref, out_ref,
              *, H, D, NB):
    f32 = jnp.float32
    bf16 = jnp.bfloat16
    HD = H * D
    F = node_ref.shape[1]
    wfcte = wfcte_ref[...]
    wproj = wproj_ref[...]
    bcol = bcol_ref[...]
    ones_row = jnp.ones((1, F), dtype=bf16)

    for j in range(NB):
        nb = node_ref[j]
        feat_ext = jax.lax.dot_general(nb, wfcte, (((1,), (0,)), ((), ())),
                                       preferred_element_type=f32)
        featb_ext = feat_ext.astype(bf16)
        featbT = jnp.transpose(featb_ext)
        el_maxs = jnp.max(feat_ext[:, HD:HD + H], axis=0,
                          keepdims=True)

        rst_rows = []
        for h in range(H):
            el_col = featb_ext[:, HD + h:HD + h + 1]
            er_row = featbT[HD + H + h:HD + H + h + 1, :]
            a = el_col + er_row
            e = jnp.maximum(a, 0.2 * a)
            t = el_maxs[0, h].astype(bf16) + er_row
            m = jnp.maximum(t, 0.2 * t)
            p = jnp.exp(e - m)
            lhs = jnp.concatenate(
                [featbT[h * D:(h + 1) * D, :], ones_row], axis=0)
            rq = jax.lax.dot_general(lhs, p, (((1,), (0,)), ((), ())),
                                     preferred_element_type=f32)
            r_row = 1.0 / rq[D:D + 1, :]
            rst_rows.append((rq[0:D, :] * r_row).astype(bf16))

        rstT = jnp.concatenate(rst_rows, axis=0)
        outT = jax.lax.dot_general(wproj, rstT, (((1,), (0,)), ((), ())),
                                   preferred_element_type=f32)
        out_ref[j] = outT + bcol


def kernel(x, W_fc, attn_l, attn_r, bias_gat, W_proj, b_proj):
    B, W, F = x.shape
    H, D = attn_l.shape
    HD = H * D

    f32 = jnp.float32
    bf16 = jnp.bfloat16
    Wfc_T = W_fc.astype(f32).T
    eye = jnp.eye(H, dtype=f32)
    Al = (attn_l.astype(f32)[:, :, None] * eye[:, None, :]).reshape(HD, H)
    Ar = (attn_r.astype(f32)[:, :, None] * eye[:, None, :]).reshape(HD, H)
    A_comb = jnp.concatenate([Al, Ar], axis=1)
    Wfc_ext = jnp.concatenate([Wfc_T, Wfc_T @ A_comb],
                              axis=1).astype(bf16)
    bcol = (W_proj.astype(f32) @ bias_gat.astype(f32)
            + b_proj.astype(f32)).reshape(W, 1)

    NB = 8
    node = jnp.transpose(x.astype(bf16), (0, 2, 1))
    body = functools.partial(_gat_body, H=H, D=D, NB=NB)
    out = pl.pallas_call(
        body,
        grid=(B // NB,),
        in_specs=[
            pl.BlockSpec((NB, F, W), lambda b: (b, 0, 0)),
            pl.BlockSpec((W, HD + 2 * H), lambda b: (0, 0)),
            pl.BlockSpec((W, HD), lambda b: (0, 0)),
            pl.BlockSpec((W, 1), lambda b: (0, 0)),
        ],
        out_specs=pl.BlockSpec((NB, W, F), lambda b: (b, 0, 0)),
        out_shape=jax.ShapeDtypeStruct((B, W, F), f32),
        compiler_params=pltpu.CompilerParams(
            dimension_semantics=("parallel",)),
    )(node, Wfc_ext, W_proj.astype(bf16), bcol)
    return out


# speedup vs baseline: 2.2005x; 1.1079x over previous
import functools

import jax
import jax.numpy as jnp
from jax.experimental import pallas as pl
from jax.experimental.pallas import tpu as pltpu


def _gat_body(node_ref, wfcte_ref, wproj_ref, bcol_ref, out_ref,
              *, H, D, NB):
    f32 = jnp.float32
    bf16 = jnp.bfloat16
    HD = H * D
    F = node_ref.shape[1]
    wfcte = wfcte_ref[...]
    wproj = wproj_ref[...]
    bcol = bcol_ref[...]
    ones_row = jnp.ones((1, F), dtype=bf16)

    for j in range(NB):
        nbT = jnp.transpose(node_ref[j].astype(bf16))
        feat_ext = jax.lax.dot_general(nbT, wfcte, (((1,), (0,)), ((), ())),
                                       preferred_element_type=f32)
        featb_ext = feat_ext.astype(bf16)
        featbT = jnp.transpose(featb_ext)
        el_maxs = jnp.max(feat_ext[:, HD:HD + H], axis=0,
                          keepdims=True)

        rst_rows = []
        for h in range(H):
            el_col = featb_ext[:, HD + h:HD + h + 1]
            er_row = featbT[HD + H + h:HD + H + h + 1, :]
            a = el_col + er_row
            e = jnp.maximum(a, 0.2 * a)
            t = el_maxs[0, h].astype(bf16) + er_row
            m = jnp.maximum(t, 0.2 * t)
            p = jnp.exp(e - m)
            lhs = jnp.concatenate(
                [featbT[h * D:(h + 1) * D, :], ones_row], axis=0)
            rq = jax.lax.dot_general(lhs, p, (((1,), (0,)), ((), ())),
                                     preferred_element_type=f32)
            r_row = 1.0 / rq[D:D + 1, :]
            rst_rows.append((rq[0:D, :] * r_row).astype(bf16))

        rstT = jnp.concatenate(rst_rows, axis=0)
        outT = jax.lax.dot_general(wproj, rstT, (((1,), (0,)), ((), ())),
                                   preferred_element_type=f32)
        out_ref[j] = outT + bcol


def kernel(x, W_fc, attn_l, attn_r, bias_gat, W_proj, b_proj):
    B, W, F = x.shape
    H, D = attn_l.shape
    HD = H * D

    f32 = jnp.float32
    bf16 = jnp.bfloat16
    Wfc_T = W_fc.astype(f32).T
    eye = jnp.eye(H, dtype=f32)
    Al = (attn_l.astype(f32)[:, :, None] * eye[:, None, :]).reshape(HD, H)
    Ar = (attn_r.astype(f32)[:, :, None] * eye[:, None, :]).reshape(HD, H)
    A_comb = jnp.concatenate([Al, Ar], axis=1)
    Wfc_ext = jnp.concatenate([Wfc_T, Wfc_T @ A_comb],
                              axis=1).astype(bf16)
    bcol = (W_proj.astype(f32) @ bias_gat.astype(f32)
            + b_proj.astype(f32)).reshape(W, 1)

    NB = 8
    body = functools.partial(_gat_body, H=H, D=D, NB=NB)
    out = pl.pallas_call(
        body,
        grid=(B // NB,),
        in_specs=[
            pl.BlockSpec((NB, W, F), lambda b: (b, 0, 0)),
            pl.BlockSpec((W, HD + 2 * H), lambda b: (0, 0)),
            pl.BlockSpec((W, HD), lambda b: (0, 0)),
            pl.BlockSpec((W, 1), lambda b: (0, 0)),
        ],
        out_specs=pl.BlockSpec((NB, W, F), lambda b: (b, 0, 0)),
        out_shape=jax.ShapeDtypeStruct((B, W, F), f32),
        compiler_params=pltpu.CompilerParams(
            dimension_semantics=("parallel",)),
    )(x, Wfc_ext, W_proj.astype(bf16), bcol)
    return out


# speedup vs baseline: 2.2120x; 1.0052x over previous
import functools

import jax
import jax.numpy as jnp
from jax.experimental import pallas as pl
from jax.experimental.pallas import tpu as pltpu


def _gat_body(node_ref, wfcte_ref, wproj_ref, bcol_ref, out_ref,
              *, H, D, NB):
    f32 = jnp.float32
    bf16 = jnp.bfloat16
    HD = H * D
    F = node_ref.shape[1]
    wfcte = wfcte_ref[...]
    wproj = wproj_ref[...]
    bcol = bcol_ref[...]
    ones_row = jnp.ones((1, F), dtype=bf16)

    for j in range(NB):
        nbT = jnp.transpose(node_ref[j].astype(bf16))
        feat_ext = jax.lax.dot_general(nbT, wfcte, (((1,), (0,)), ((), ())),
                                       preferred_element_type=f32)
        featb_ext = feat_ext.astype(bf16)
        featbT = jnp.transpose(featb_ext)
        el_maxs = jnp.max(feat_ext[:, HD:HD + H], axis=0,
                          keepdims=True)

        rst_rows = []
        for h in range(H):
            el_col = featb_ext[:, HD + h:HD + h + 1]
            er_row = featbT[HD + H + h:HD + H + h + 1, :]
            a = el_col + er_row
            e = jnp.maximum(a, 0.2 * a)
            t = el_maxs[0, h].astype(bf16) + er_row
            m = jnp.maximum(t, 0.2 * t)
            p = jnp.exp(e - m)
            lhs = jnp.concatenate(
                [featbT[h * D:(h + 1) * D, :], ones_row], axis=0)
            rq = jax.lax.dot_general(lhs, p, (((1,), (0,)), ((), ())),
                                     preferred_element_type=f32)
            r_row = 1.0 / rq[D:D + 1, :]
            rst_rows.append((rq[0:D, :] * r_row).astype(bf16))

        rstT = jnp.concatenate(rst_rows, axis=0)
        outT = jax.lax.dot_general(wproj, rstT, (((1,), (0,)), ((), ())),
                                   preferred_element_type=f32)
        out_ref[j] = outT + bcol


def kernel(x, W_fc, attn_l, attn_r, bias_gat, W_proj, b_proj):
    B, W, F = x.shape
    H, D = attn_l.shape
    HD = H * D

    f32 = jnp.float32
    bf16 = jnp.bfloat16
    Wfc_T = W_fc.astype(f32).T
    eye = jnp.eye(H, dtype=f32)
    Al = (attn_l.astype(f32)[:, :, None] * eye[:, None, :]).reshape(HD, H)
    Ar = (attn_r.astype(f32)[:, :, None] * eye[:, None, :]).reshape(HD, H)
    A_comb = jnp.concatenate([Al, Ar], axis=1)
    Wfc_ext = jnp.concatenate([Wfc_T, Wfc_T @ A_comb],
                              axis=1).astype(bf16)
    bcol = (W_proj.astype(f32) @ bias_gat.astype(f32)
            + b_proj.astype(f32)).reshape(W, 1)

    NB = 16
    body = functools.partial(_gat_body, H=H, D=D, NB=NB)
    out = pl.pallas_call(
        body,
        grid=(B // NB,),
        in_specs=[
            pl.BlockSpec((NB, W, F), lambda b: (b, 0, 0)),
            pl.BlockSpec((W, HD + 2 * H), lambda b: (0, 0)),
            pl.BlockSpec((W, HD), lambda b: (0, 0)),
            pl.BlockSpec((W, 1), lambda b: (0, 0)),
        ],
        out_specs=pl.BlockSpec((NB, W, F), lambda b: (b, 0, 0)),
        out_shape=jax.ShapeDtypeStruct((B, W, F), f32),
        compiler_params=pltpu.CompilerParams(
            dimension_semantics=("parallel",)),
    )(x, Wfc_ext, W_proj.astype(bf16), bcol)
    return out
